# Initial kernel scaffold; baseline (speedup 1.0000x reference)
import jax
import jax.numpy as jnp
from jax.experimental import pallas as pl


def kernel(inputs, token_table, pos_table):
    raise NotImplementedError("write your pallas kernel here")


# speedup vs baseline: 2.2500x; 2.2500x over previous
import functools

import jax
import jax.numpy as jnp
from jax import lax
from jax.experimental import pallas as pl
from jax.experimental.pallas import tpu as pltpu
from jax.experimental.pallas import tpu_sc as plsc

BATCH = 1024
SEQ = 200
EMBED = 128
HALF = 100
NC = 2
NS = 16
NW = NC * NS
SEQ_PER_W = BATCH // NW
NV = EMBED // 16


def _emb_body(idx_hbm, tok_hbm, pos_hbm, out_hbm, idx_v, rows_v, pos_v, sem):
    wid = lax.axis_index("s") * NC + lax.axis_index("c")

    pltpu.sync_copy(pos_hbm, pos_v)

    def per_seq(i, _):
        seq = wid * SEQ_PER_W + i
        pltpu.sync_copy(idx_hbm.at[seq], idx_v)
        cp0 = pltpu.async_copy(tok_hbm.at[idx_v.at[0]], rows_v.at[0], sem)
        cp1 = pltpu.async_copy(tok_hbm.at[idx_v.at[1]], rows_v.at[1], sem)
        cp0.wait()
        cp1.wait()

        def add_row(r, _):
            for h in range(2):
                for j in range(NV):
                    sl = pl.ds(j * 16, 16)
                    rows_v[h, r, sl] = rows_v[h, r, sl] + pos_v[h, r, sl]
            return ()

        lax.fori_loop(0, HALF, add_row, ())
        pltpu.sync_copy(rows_v, out_hbm.at[seq])
        return ()

    lax.fori_loop(0, SEQ_PER_W, per_seq, ())


@jax.jit
def kernel(inputs, token_table, pos_table):
    idx = inputs.reshape(BATCH, 2, HALF).astype(jnp.int32)
    pos = pos_table.reshape(2, HALF, EMBED)
    mesh = plsc.VectorSubcoreMesh(core_axis_name="c", subcore_axis_name="s")
    run = pl.kernel(
        _emb_body,
        out_type=jax.ShapeDtypeStruct((BATCH, 2, HALF, EMBED), jnp.float32),
        mesh=mesh,
        scratch_types=[
            pltpu.VMEM((2, HALF), jnp.int32),
            pltpu.VMEM((2, HALF, EMBED), jnp.float32),
            pltpu.VMEM((2, HALF, EMBED), jnp.float32),
            pltpu.SemaphoreType.DMA,
        ],
    )
    out = run(idx, token_table, pos)
    return out.reshape(BATCH, SEQ, EMBED)


# speedup vs baseline: 5.5077x; 2.4479x over previous
import jax
import jax.numpy as jnp
from jax import lax
from jax.experimental import pallas as pl
from jax.experimental.pallas import tpu as pltpu
from jax.experimental.pallas import tpu_sc as plsc

BATCH = 1024
SEQ = 200
EMBED = 128
HALF = 100
NC = 2
NS = 16
NW = NC * NS
SEQ_PER_W = BATCH // NW
NV = EMBED // 16
NBUF = 2


def _emb_body(idx_hbm, tok_hbm, pos_hbm, out_hbm,
              idx_v, rows_v, pos_v, gsem0, gsem1, ssem0, ssem1):
    wid = lax.axis_index("s") * NC + lax.axis_index("c")
    base_seq = wid * SEQ_PER_W
    gsems = (gsem0, gsem1)
    ssems = (ssem0, ssem1)

    pltpu.sync_copy(pos_hbm, pos_v)

    def gather_descs(b, issue):
        mk = pltpu.async_copy if issue else pltpu.make_async_copy
        c0 = mk(tok_hbm.at[idx_v.at[b, 0]], rows_v.at[b, pl.ds(0, HALF)],
                gsems[b])
        c1 = mk(tok_hbm.at[idx_v.at[b, 1]], rows_v.at[b, pl.ds(HALF, HALF)],
                gsems[b])
        return c0, c1

    def launch(i, b):
        pltpu.sync_copy(idx_hbm.at[base_seq + i], idx_v.at[b])
        gather_descs(b, issue=True)

    def wait_gather(b):
        for c in gather_descs(b, issue=False):
            c.wait()

    def start_store(i, b):
        pltpu.async_copy(rows_v.at[b], out_hbm.at[base_seq + i], ssems[b])

    def wait_store(i, b):
        pltpu.make_async_copy(rows_v.at[b], out_hbm.at[base_seq + i],
                              ssems[b]).wait()

    def add_pos(b):
        def body(r, _):
            for u in range(2):
                rr = r * 2 + u
                for j in range(NV):
                    sl = pl.ds(j * 16, 16)
                    rows_v[b, rr, sl] = rows_v[b, rr, sl] + pos_v[rr, sl]
            return ()
        lax.fori_loop(0, SEQ // 2, body, ())

    launch(0, 0)

    def outer(o, _):
        for b in range(NBUF):
            i = o * NBUF + b
            bn = 1 - b

            @pl.when(i + 1 < SEQ_PER_W)
            def _():
                @pl.when(i >= 1)
                def _():
                    wait_store(i - 1, bn)
                launch(i + 1, bn)

            wait_gather(b)
            add_pos(b)
            start_store(i, b)
        return ()

    lax.fori_loop(0, SEQ_PER_W // NBUF, outer, ())
    wait_store(SEQ_PER_W - 2, 0)
    wait_store(SEQ_PER_W - 1, 1)


@jax.jit
def kernel(inputs, token_table, pos_table):
    idx = inputs.reshape(BATCH, 2, HALF).astype(jnp.int32)
    mesh = plsc.VectorSubcoreMesh(core_axis_name="c", subcore_axis_name="s")
    run = pl.kernel(
        _emb_body,
        out_type=jax.ShapeDtypeStruct((BATCH, SEQ, EMBED), jnp.float32),
        mesh=mesh,
        scratch_types=[
            pltpu.VMEM((NBUF, 2, HALF), jnp.int32),
            pltpu.VMEM((NBUF, SEQ, EMBED), jnp.float32),
            pltpu.VMEM((SEQ, EMBED), jnp.float32),
            pltpu.SemaphoreType.DMA,
            pltpu.SemaphoreType.DMA,
            pltpu.SemaphoreType.DMA,
            pltpu.SemaphoreType.DMA,
        ],
    )
    return run(idx, token_table, pos_table)
